# Initial kernel scaffold; baseline (speedup 1.0000x reference)
import jax
import jax.numpy as jnp
from jax.experimental import pallas as pl


def kernel(x, edge_index, edge_attr, batch, W_enc, b_enc, eps, We, bee, W1, b1, g1, bb1, W2, b2, g2, bb2):
    raise NotImplementedError("write your pallas kernel here")


# speedup vs baseline: 1.6442x; 1.6442x over previous
import functools

import jax
import jax.numpy as jnp
from jax import lax
from jax.experimental import pallas as pl
from jax.experimental.pallas import tpu as pltpu
from jax.experimental.pallas import tpu_sc as plsc

N = 10000
E = 320000
D = 128
DE = 16
L = 3

NC = 2
NS = 16
LANES = 16
NW = NC * NS

B = 128
CHUNKS = 79
EPW = B * CHUNKS
EP = EPW * NW
PADE = EP - E
NPAD = 10112
RPT = NPAD // NS

_sc_mesh = plsc.VectorSubcoreMesh(
    core_axis_name="c", subcore_axis_name="s", num_cores=NC, num_subcores=NS)


@functools.partial(
    pl.kernel,
    out_type=jax.ShapeDtypeStruct((NC, NPAD, D), jnp.float32),
    mesh=_sc_mesh,
    scratch_types=[
        pltpu.VMEM((B,), jnp.int32),
        pltpu.VMEM((B,), jnp.int32),
        pltpu.VMEM((B, D), jnp.float32),
        pltpu.VMEM((B, D), jnp.float32),
        pltpu.VMEM_SHARED((NPAD, D), jnp.float32),
        pltpu.SemaphoreType.DMA,
    ],
)
def _edge_stage(h_hbm, e_hbm, src_hbm, dst_hbm, out_hbm,
                srcv, dstv, hbuf, ebuf, agg_sh, sem):
    c = lax.axis_index("c")
    s = lax.axis_index("s")
    wid = c * NS + s

    def _zrow(r, carry):
        for cc in range(D // LANES):
            hbuf[r, pl.ds(cc * LANES, LANES)] = jnp.zeros((LANES,), jnp.float32)
        return carry
    lax.fori_loop(0, B, _zrow, 0)
    r0 = s * RPT
    for k in range(RPT // B):
        pltpu.sync_copy(hbuf, agg_sh.at[pl.ds(r0 + k * B, B)])
    rem = RPT % B
    if rem:
        pltpu.sync_copy(hbuf.at[pl.ds(0, rem)],
                        agg_sh.at[pl.ds(r0 + (RPT // B) * B, rem)])
    plsc.subcore_barrier()

    def _chunk(i, carry):
        base = wid * EPW + i * B
        pltpu.sync_copy(src_hbm.at[pl.ds(base, B)], srcv)
        pltpu.sync_copy(dst_hbm.at[pl.ds(base, B)], dstv)
        pltpu.sync_copy(e_hbm.at[pl.ds(base, B)], ebuf)
        pltpu.async_copy(h_hbm.at[srcv], hbuf, sem).wait()

        def _row(r, rc):
            for cc in range(D // LANES):
                sl = pl.ds(cc * LANES, LANES)
                ebuf[r, sl] = jnp.maximum(hbuf[r, sl] + ebuf[r, sl], 0.0)
            return rc
        lax.fori_loop(0, B, _row, 0)
        pltpu.sync_copy(ebuf, agg_sh.at[dstv], add=True)
        return carry
    lax.fori_loop(0, CHUNKS, _chunk, 0)
    plsc.subcore_barrier()

    for k in range(RPT // B):
        pltpu.sync_copy(agg_sh.at[pl.ds(r0 + k * B, B)], hbuf)
        pltpu.sync_copy(hbuf, out_hbm.at[c, pl.ds(r0 + k * B, B)])
    if rem:
        pltpu.sync_copy(agg_sh.at[pl.ds(r0 + (RPT // B) * B, rem)],
                        hbuf.at[pl.ds(0, rem)])
        pltpu.sync_copy(hbuf.at[pl.ds(0, rem)],
                        out_hbm.at[c, pl.ds(r0 + (RPT // B) * B, rem)])


def _enc_body(x_ref, w_ref, b_ref, o_ref):
    o_ref[...] = jnp.dot(x_ref[...], w_ref[...],
                         preferred_element_type=jnp.float32) + b_ref[...]


BE = 1024


def _eenc_body(ea_ref, we_ref, be_ref, o_ref):
    o_ref[0] = jnp.dot(ea_ref[...], we_ref[0],
                       preferred_element_type=jnp.float32) + be_ref[0]


def _node_body(h_ref, a_ref, eps_ref, w1_ref, b1_ref, g1_ref, bb1_ref,
               w2_ref, b2_ref, g2_ref, bb2_ref, o_ref, *, last):
    agg = a_ref[0, :N, :] + a_ref[1, :N, :]
    z = (1.0 + eps_ref[0, 0]) * h_ref[...] + agg
    z = jnp.dot(z, w1_ref[...], preferred_element_type=jnp.float32) + b1_ref[...]
    m = jnp.mean(z, axis=0, keepdims=True)
    v = jnp.mean((z - m) ** 2, axis=0, keepdims=True)
    z = g1_ref[...] * (z - m) * lax.rsqrt(v + 1e-5) + bb1_ref[...]
    z = jnp.maximum(z, 0.0)
    z = jnp.dot(z, w2_ref[...], preferred_element_type=jnp.float32) + b2_ref[...]
    m2 = jnp.mean(z, axis=0, keepdims=True)
    v2 = jnp.mean((z - m2) ** 2, axis=0, keepdims=True)
    z = g2_ref[...] * (z - m2) * lax.rsqrt(v2 + 1e-5) + bb2_ref[...]
    if not last:
        z = jnp.maximum(z, 0.0)
    o_ref[...] = z


def kernel(x, edge_index, edge_attr, batch, W_enc, b_enc, eps, We, bee,
           W1, b1, g1, bb1, W2, b2, g2, bb2):
    src = jnp.concatenate([edge_index[0], jnp.zeros((PADE,), jnp.int32)])
    dst = jnp.concatenate([edge_index[1], jnp.full((PADE,), N, jnp.int32)])
    ea = jnp.concatenate([edge_attr, jnp.zeros((PADE, DE), jnp.float32)], axis=0)

    h = pl.pallas_call(
        _enc_body,
        out_shape=jax.ShapeDtypeStruct((N, D), jnp.float32),
    )(x, W_enc, b_enc.reshape(1, D))

    e_all = pl.pallas_call(
        _eenc_body,
        grid=(L, EP // BE),
        in_specs=[
            pl.BlockSpec((BE, DE), lambda l, i: (i, 0)),
            pl.BlockSpec((1, DE, D), lambda l, i: (l, 0, 0)),
            pl.BlockSpec((1, 1, D), lambda l, i: (l, 0, 0)),
        ],
        out_specs=pl.BlockSpec((1, BE, D), lambda l, i: (l, i, 0)),
        out_shape=jax.ShapeDtypeStruct((L, EP, D), jnp.float32),
    )(ea, We, bee.reshape(L, 1, D))

    for l in range(L):
        agg2 = _edge_stage(h, e_all[l], src, dst)
        h = pl.pallas_call(
            functools.partial(_node_body, last=(l == L - 1)),
            out_shape=jax.ShapeDtypeStruct((N, D), jnp.float32),
        )(h, agg2, eps[l].reshape(1, 1),
          W1[l], b1[l].reshape(1, 2 * D), g1[l].reshape(1, 2 * D),
          bb1[l].reshape(1, 2 * D),
          W2[l], b2[l].reshape(1, D), g2[l].reshape(1, D),
          bb2[l].reshape(1, D))
    return h
